# Initial kernel scaffold; baseline (speedup 1.0000x reference)
import jax
import jax.numpy as jnp
from jax.experimental import pallas as pl


def kernel(x, edge_index, edge_attr):
    raise NotImplementedError("write your pallas kernel here")


# speedup vs baseline: 2.5186x; 2.5186x over previous
import functools

import jax
import jax.numpy as jnp
from jax import lax
from jax.experimental import pallas as pl
from jax.experimental.pallas import tpu as pltpu
from jax.experimental.pallas import tpu_sc as plsc

E = 320000
D = 128
A = 16
NC = 2
NS = 16
NW = NC * NS
EPW = E // NW
CH = 80
NCHUNK = EPW // CH

_mesh = plsc.VectorSubcoreMesh(core_axis_name="c", subcore_axis_name="s")


@functools.partial(
    pl.kernel,
    out_type=jax.ShapeDtypeStruct((E, 2 * D + A), jnp.float32),
    mesh=_mesh,
    scratch_types=[
        pltpu.VMEM((NCHUNK, CH), jnp.int32),
        pltpu.VMEM((NCHUNK, CH), jnp.int32),
        pltpu.VMEM((CH, D), jnp.float32),
        pltpu.VMEM((CH, D), jnp.float32),
        pltpu.VMEM((CH, A), jnp.float32),
        pltpu.SemaphoreType.DMA,
    ],
)
def _edge_concat(x_hbm, ei_hbm, ea_hbm, out_hbm,
                 sidx, didx, sbuf, dbuf, abuf, sem):
    wid = lax.axis_index("s") * NC + lax.axis_index("c")
    base = wid * EPW

    pltpu.sync_copy(ei_hbm.at[0, wid], sidx)
    pltpu.sync_copy(ei_hbm.at[1, wid], didx)

    def body(j, carry):
        gbase = base + j * CH
        c1 = pltpu.async_copy(x_hbm.at[sidx.at[j]], sbuf, sem)
        c2 = pltpu.async_copy(x_hbm.at[didx.at[j]], dbuf, sem)
        c3 = pltpu.async_copy(ea_hbm.at[pl.ds(gbase, CH)], abuf, sem)
        c1.wait()
        c2.wait()
        c3.wait()
        pltpu.sync_copy(sbuf, out_hbm.at[pl.ds(gbase, CH), pl.ds(0, D)])
        pltpu.sync_copy(dbuf, out_hbm.at[pl.ds(gbase, CH), pl.ds(D, D)])
        pltpu.sync_copy(abuf, out_hbm.at[pl.ds(gbase, CH), pl.ds(2 * D, A)])
        return carry

    lax.fori_loop(0, NCHUNK, body, 0)


def kernel(x, edge_index, edge_attr):
    ei = edge_index.astype(jnp.int32).reshape(2, NW, NCHUNK, CH)
    return _edge_concat(x, ei, edge_attr)
